# baseline (speedup 1.0000x reference)
import jax
import jax.numpy as jnp
from jax import lax
from jax.experimental import pallas as pl
from jax.experimental.pallas import tpu as pltpu
from jax.experimental.pallas import tpu_sc as plsc

G = 8192
D = 256
K = 5
BM = 128
BN = 512
T = G // BN

B = 65536
_NC, _NS = 2, 16
_NW = _NC * _NS
_BPW = B // _NW
_CH = 128
_NCH = _BPW // _CH


def _knn_table_kernel(x_ref, full_ref, out_ref, dist_ref, sqrow_ref,
                      hi_ref, lo_ref):
    pid = pl.program_id(0)

    @pl.when(pid == 0)
    def _():
        ones_row = jnp.ones((1, D), jnp.float32)
        for t in range(T):
            xt = full_ref[pl.ds(t * BN, BN), :]
            sqrow_ref[:, pl.ds(t * BN, BN)] = lax.dot_general(
                ones_row, xt * xt, (((1,), (1,)), ((), ())),
                precision=lax.Precision.HIGHEST,
                preferred_element_type=jnp.float32)
            hi = xt.astype(jnp.bfloat16)
            hi_ref[pl.ds(t * BN, BN), :] = hi
            lo_ref[pl.ds(t * BN, BN), :] = (
                xt - hi.astype(jnp.float32)).astype(jnp.bfloat16)

    x = x_ref[...]
    col_base = lax.broadcasted_iota(jnp.int32, (BM, BN), 1)
    inf = jnp.float32(jnp.inf)
    row_id = pid * BM + lax.broadcasted_iota(jnp.int32, (BM, 1), 0)
    diag_t = (pid * BM) // BN
    for t in range(T):
        xt = full_ref[pl.ds(t * BN, BN), :]
        dotv = lax.dot_general(
            x, xt, (((1,), (1,)), ((), ())),
            preferred_element_type=jnp.float32)
        sqb = sqrow_ref[:, pl.ds(t * BN, BN)]
        val = sqb - 2.0 * dotv

        @pl.when(diag_t == t)
        def _(val=val, t=t):
            dist_ref[:, pl.ds(t * BN, BN)] = jnp.where(
                col_base + t * BN == row_id, inf, val)

        @pl.when(diag_t != t)
        def _(val=val, t=t):
            dist_ref[:, pl.ds(t * BN, BN)] = val

    colf_base = col_base.astype(jnp.float32)
    big = jnp.float32(3e8)
    sel = []
    for _ in range(K - 1):
        bestv = jnp.full((BM, 1), inf, jnp.float32)
        besti = jnp.full((BM, 1), big, jnp.float32)
        for t in range(T):
            d = dist_ref[:, pl.ds(t * BN, BN)]
            colf = colf_base + jnp.float32(t * BN)
            for s in sel:
                d = jnp.where(colf == s, inf, d)
            tmin = jnp.min(d, axis=1, keepdims=True)
            targ = jnp.min(jnp.where(d == tmin, colf, big),
                           axis=1, keepdims=True)
            take = tmin < bestv
            bestv = jnp.where(take, tmin, bestv)
            besti = jnp.where(take, targ, besti)
        sel.append(besti)

    acc = jnp.zeros((BM, D), jnp.float32)
    for t in range(T):
        ds = pl.ds(t * BN, BN)
        colf = colf_base + jnp.float32(t * BN)
        oh = ((colf == sel[0]) | (colf == sel[1]) |
              (colf == sel[2]) | (colf == sel[3])).astype(jnp.bfloat16)
        acc = (acc
               + lax.dot_general(oh, hi_ref[ds, :], (((1,), (0,)), ((), ())),
                                 preferred_element_type=jnp.float32)
               + lax.dot_general(oh, lo_ref[ds, :], (((1,), (0,)), ((), ())),
                                 preferred_element_type=jnp.float32))

    s = jnp.sum(acc * acc, axis=1, keepdims=True)
    out_ref[...] = acc / jnp.maximum(jnp.sqrt(s), 1e-12)


def _build_table(x):
    return pl.pallas_call(
        _knn_table_kernel,
        grid=(G // BM,),
        in_specs=[
            pl.BlockSpec((BM, D), lambda i: (i, 0)),
            pl.BlockSpec((G, D), lambda i: (0, 0)),
        ],
        out_specs=pl.BlockSpec((BM, D), lambda i: (i, 0)),
        out_shape=jax.ShapeDtypeStruct((G, D), jnp.float32),
        scratch_shapes=[
            pltpu.VMEM((BM, G), jnp.float32),
            pltpu.VMEM((1, G), jnp.float32),
            pltpu.VMEM((G, D), jnp.bfloat16),
            pltpu.VMEM((G, D), jnp.bfloat16),
        ],
    )(x, x)


def _gather_body(table_hbm, idx_hbm, out_hbm, idx_v, buf0, buf1, sem0, sem1):
    wid = lax.axis_index("s") * _NC + lax.axis_index("c")
    base = wid * _BPW
    pltpu.sync_copy(idx_hbm.at[pl.ds(base, _BPW)], idx_v)
    bufs = (buf0, buf1)
    sems = (sem0, sem1)
    handles = [None, None]
    handles[0] = pltpu.async_copy(
        table_hbm.at[idx_v.at[pl.ds(0, _CH)]], bufs[0], sems[0])
    for c in range(_NCH):
        cur = c % 2
        if c + 1 < _NCH:
            nxt = (c + 1) % 2
            handles[nxt] = pltpu.async_copy(
                table_hbm.at[idx_v.at[pl.ds((c + 1) * _CH, _CH)]],
                bufs[nxt], sems[nxt])
        handles[cur].wait()
        pltpu.sync_copy(bufs[cur], out_hbm.at[pl.ds(base + c * _CH, _CH)])


def _gather_rows(table, idx):
    mesh = plsc.VectorSubcoreMesh(core_axis_name="c", subcore_axis_name="s")
    f = pl.kernel(
        _gather_body,
        out_type=jax.ShapeDtypeStruct((B, D), jnp.float32),
        mesh=mesh,
        scratch_types=[
            pltpu.VMEM((_BPW,), jnp.int32),
            pltpu.VMEM((_CH, D), jnp.float32),
            pltpu.VMEM((_CH, D), jnp.float32),
            pltpu.SemaphoreType.DMA,
            pltpu.SemaphoreType.DMA,
        ],
    )
    return f(table, idx)


def kernel(graph_emb, batch, k):
    table = _build_table(graph_emb)
    return _gather_rows(table, batch.astype(jnp.int32))


# speedup vs baseline: 1.3371x; 1.3371x over previous
import jax
import jax.numpy as jnp
from jax import lax
from jax.experimental import pallas as pl
from jax.experimental.pallas import tpu as pltpu
from jax.experimental.pallas import tpu_sc as plsc

G = 8192
D = 256
K = 5
BM = 128
BN = 512
T = G // BN

B = 65536
_NC, _NS = 2, 16
_NW = _NC * _NS
_BPW = B // _NW
_CH = 128
_NCH = _BPW // _CH


def _knn_table_kernel(x_ref, full_ref, out_ref, dist_ref, sqrow_ref,
                      hi_ref, lo_ref):
    pid = pl.program_id(0)

    @pl.when(pid == 0)
    def _():
        ones_row = jnp.ones((1, D), jnp.float32)
        for t in range(T):
            xt = full_ref[pl.ds(t * BN, BN), :]
            sqrow_ref[:, pl.ds(t * BN, BN)] = lax.dot_general(
                ones_row, xt * xt, (((1,), (1,)), ((), ())),
                precision=lax.Precision.HIGHEST,
                preferred_element_type=jnp.float32)
            hi = xt.astype(jnp.bfloat16)
            hi_ref[pl.ds(t * BN, BN), :] = hi
            lo_ref[pl.ds(t * BN, BN), :] = (
                xt - hi.astype(jnp.float32)).astype(jnp.bfloat16)

    x = x_ref[...]
    col_base = lax.broadcasted_iota(jnp.int32, (BM, BN), 1)
    inf = jnp.float32(jnp.inf)
    row_id = pid * BM + lax.broadcasted_iota(jnp.int32, (BM, 1), 0)
    for t in range(T):
        xt = full_ref[pl.ds(t * BN, BN), :]
        dotv = lax.dot_general(
            x, xt, (((1,), (1,)), ((), ())),
            preferred_element_type=jnp.float32)
        sqb = sqrow_ref[:, pl.ds(t * BN, BN)]
        col = col_base + t * BN
        dist_ref[:, pl.ds(t * BN, BN)] = jnp.where(
            col == row_id, inf, sqb - 2.0 * dotv)

    colf_base = col_base.astype(jnp.float32)
    big = jnp.float32(3e8)
    sel = []
    for _ in range(K - 1):
        bestv = jnp.full((BM, 1), inf, jnp.float32)
        besti = jnp.full((BM, 1), big, jnp.float32)
        for t in range(T):
            d = dist_ref[:, pl.ds(t * BN, BN)]
            colf = colf_base + jnp.float32(t * BN)
            for s in sel:
                d = jnp.where(colf == s, inf, d)
            tmin = jnp.min(d, axis=1, keepdims=True)
            targ = jnp.min(jnp.where(d == tmin, colf, big),
                           axis=1, keepdims=True)
            take = tmin < bestv
            bestv = jnp.where(take, tmin, bestv)
            besti = jnp.where(take, targ, besti)
        sel.append(besti)

    acc = jnp.zeros((BM, D), jnp.float32)
    for t in range(T):
        ds = pl.ds(t * BN, BN)
        colf = colf_base + jnp.float32(t * BN)
        oh = ((colf == sel[0]) | (colf == sel[1]) |
              (colf == sel[2]) | (colf == sel[3])).astype(jnp.bfloat16)
        acc = (acc
               + lax.dot_general(oh, hi_ref[ds, :], (((1,), (0,)), ((), ())),
                                 preferred_element_type=jnp.float32)
               + lax.dot_general(oh, lo_ref[ds, :], (((1,), (0,)), ((), ())),
                                 preferred_element_type=jnp.float32))

    s = jnp.sum(acc * acc, axis=1, keepdims=True)
    out_ref[...] = acc / jnp.maximum(jnp.sqrt(s), 1e-12)


def _build_table(x):
    return pl.pallas_call(
        _knn_table_kernel,
        grid=(G // BM,),
        in_specs=[
            pl.BlockSpec((BM, D), lambda i: (i, 0)),
            pl.BlockSpec((G, D), lambda i: (0, 0)),
        ],
        out_specs=pl.BlockSpec((BM, D), lambda i: (i, 0)),
        out_shape=jax.ShapeDtypeStruct((G, D), jnp.float32),
        scratch_shapes=[
            pltpu.VMEM((BM, G), jnp.float32),
            pltpu.VMEM((1, G), jnp.float32),
            pltpu.VMEM((G, D), jnp.bfloat16),
            pltpu.VMEM((G, D), jnp.bfloat16),
        ],
    )(x, x)


def _gather_body(table_hbm, idx_hbm, out_hbm, idx_v, buf0, buf1, sem0, sem1):
    wid = lax.axis_index("s") * _NC + lax.axis_index("c")
    base = wid * _BPW
    pltpu.sync_copy(idx_hbm.at[pl.ds(base, _BPW)], idx_v)
    bufs = (buf0, buf1)
    sems = (sem0, sem1)
    handles = [None, None]
    handles[0] = pltpu.async_copy(
        table_hbm.at[idx_v.at[pl.ds(0, _CH)]], bufs[0], sems[0])
    for c in range(_NCH):
        cur = c % 2
        if c + 1 < _NCH:
            nxt = (c + 1) % 2
            handles[nxt] = pltpu.async_copy(
                table_hbm.at[idx_v.at[pl.ds((c + 1) * _CH, _CH)]],
                bufs[nxt], sems[nxt])
        handles[cur].wait()
        pltpu.sync_copy(bufs[cur], out_hbm.at[pl.ds(base + c * _CH, _CH)])


def _gather_rows(table, idx):
    mesh = plsc.VectorSubcoreMesh(core_axis_name="c", subcore_axis_name="s")
    f = pl.kernel(
        _gather_body,
        out_type=jax.ShapeDtypeStruct((B, D), jnp.float32),
        mesh=mesh,
        scratch_types=[
            pltpu.VMEM((_BPW,), jnp.int32),
            pltpu.VMEM((_CH, D), jnp.float32),
            pltpu.VMEM((_CH, D), jnp.float32),
            pltpu.SemaphoreType.DMA,
            pltpu.SemaphoreType.DMA,
        ],
    )
    return f(table, idx)


def kernel(graph_emb, batch, k):
    table = _build_table(graph_emb)
    return _gather_rows(table, batch.astype(jnp.int32))


# speedup vs baseline: 1.5922x; 1.1908x over previous
import jax
import jax.numpy as jnp
from jax import lax
from jax.experimental import pallas as pl
from jax.experimental.pallas import tpu as pltpu
from jax.experimental.pallas import tpu_sc as plsc

G = 8192
D = 256
K = 5
BM = 256
BN = 512
T = G // BN

B = 65536
_NC, _NS = 2, 16
_NW = _NC * _NS
_BPW = B // _NW
_CH = 128
_NCH = _BPW // _CH


def _knn_table_kernel(x_ref, full_ref, out_ref, dist_ref, sqrow_ref,
                      hi_ref, lo_ref):
    pid = pl.program_id(0)

    @pl.when(pid == 0)
    def _():
        ones_row = jnp.ones((1, D), jnp.float32)
        for t in range(T):
            xt = full_ref[pl.ds(t * BN, BN), :]
            sqrow_ref[:, pl.ds(t * BN, BN)] = lax.dot_general(
                ones_row, xt * xt, (((1,), (1,)), ((), ())),
                precision=lax.Precision.HIGHEST,
                preferred_element_type=jnp.float32)
            hi = xt.astype(jnp.bfloat16)
            hi_ref[pl.ds(t * BN, BN), :] = hi
            lo_ref[pl.ds(t * BN, BN), :] = (
                xt - hi.astype(jnp.float32)).astype(jnp.bfloat16)

    x = x_ref[...]
    col_base = lax.broadcasted_iota(jnp.int32, (BM, BN), 1)
    inf = jnp.float32(jnp.inf)
    row_id = pid * BM + lax.broadcasted_iota(jnp.int32, (BM, 1), 0)
    for t in range(T):
        xt = full_ref[pl.ds(t * BN, BN), :]
        dotv = lax.dot_general(
            x, xt, (((1,), (1,)), ((), ())),
            preferred_element_type=jnp.float32)
        sqb = sqrow_ref[:, pl.ds(t * BN, BN)]
        col = col_base + t * BN
        dist_ref[:, pl.ds(t * BN, BN)] = jnp.where(
            col == row_id, inf, sqb - 2.0 * dotv)

    colf_base = col_base.astype(jnp.float32)
    big = jnp.float32(3e8)
    sel = []
    for _ in range(K - 1):
        bestv = jnp.full((BM, 1), inf, jnp.float32)
        besti = jnp.full((BM, 1), big, jnp.float32)
        for t in range(T):
            d = dist_ref[:, pl.ds(t * BN, BN)]
            colf = colf_base + jnp.float32(t * BN)
            for s in sel:
                d = jnp.where(colf == s, inf, d)
            tmin = jnp.min(d, axis=1, keepdims=True)
            targ = jnp.min(jnp.where(d == tmin, colf, big),
                           axis=1, keepdims=True)
            take = tmin < bestv
            bestv = jnp.where(take, tmin, bestv)
            besti = jnp.where(take, targ, besti)
        sel.append(besti)

    acc = jnp.zeros((BM, D), jnp.float32)
    for t in range(T):
        ds = pl.ds(t * BN, BN)
        colf = colf_base + jnp.float32(t * BN)
        oh = ((colf == sel[0]) | (colf == sel[1]) |
              (colf == sel[2]) | (colf == sel[3])).astype(jnp.bfloat16)
        acc = (acc
               + lax.dot_general(oh, hi_ref[ds, :], (((1,), (0,)), ((), ())),
                                 preferred_element_type=jnp.float32)
               + lax.dot_general(oh, lo_ref[ds, :], (((1,), (0,)), ((), ())),
                                 preferred_element_type=jnp.float32))

    s = jnp.sum(acc * acc, axis=1, keepdims=True)
    out_ref[...] = acc / jnp.maximum(jnp.sqrt(s), 1e-12)


def _build_table(x):
    return pl.pallas_call(
        _knn_table_kernel,
        grid=(G // BM,),
        in_specs=[
            pl.BlockSpec((BM, D), lambda i: (i, 0)),
            pl.BlockSpec((G, D), lambda i: (0, 0)),
        ],
        out_specs=pl.BlockSpec((BM, D), lambda i: (i, 0)),
        out_shape=jax.ShapeDtypeStruct((G, D), jnp.float32),
        scratch_shapes=[
            pltpu.VMEM((BM, G), jnp.float32),
            pltpu.VMEM((1, G), jnp.float32),
            pltpu.VMEM((G, D), jnp.bfloat16),
            pltpu.VMEM((G, D), jnp.bfloat16),
        ],
    )(x, x)


def _gather_body(table_hbm, idx_hbm, out_hbm, idx_v, buf0, buf1, sem0, sem1):
    wid = lax.axis_index("s") * _NC + lax.axis_index("c")
    base = wid * _BPW
    pltpu.sync_copy(idx_hbm.at[pl.ds(base, _BPW)], idx_v)
    bufs = (buf0, buf1)
    sems = (sem0, sem1)
    handles = [None, None]
    handles[0] = pltpu.async_copy(
        table_hbm.at[idx_v.at[pl.ds(0, _CH)]], bufs[0], sems[0])
    for c in range(_NCH):
        cur = c % 2
        if c + 1 < _NCH:
            nxt = (c + 1) % 2
            handles[nxt] = pltpu.async_copy(
                table_hbm.at[idx_v.at[pl.ds((c + 1) * _CH, _CH)]],
                bufs[nxt], sems[nxt])
        handles[cur].wait()
        pltpu.sync_copy(bufs[cur], out_hbm.at[pl.ds(base + c * _CH, _CH)])


def _gather_rows(table, idx):
    mesh = plsc.VectorSubcoreMesh(core_axis_name="c", subcore_axis_name="s")
    f = pl.kernel(
        _gather_body,
        out_type=jax.ShapeDtypeStruct((B, D), jnp.float32),
        mesh=mesh,
        scratch_types=[
            pltpu.VMEM((_BPW,), jnp.int32),
            pltpu.VMEM((_CH, D), jnp.float32),
            pltpu.VMEM((_CH, D), jnp.float32),
            pltpu.SemaphoreType.DMA,
            pltpu.SemaphoreType.DMA,
        ],
    )
    return f(table, idx)


def kernel(graph_emb, batch, k):
    table = _build_table(graph_emb)
    return _gather_rows(table, batch.astype(jnp.int32))


# speedup vs baseline: 1.6337x; 1.0260x over previous
import jax
import jax.numpy as jnp
from jax import lax
from jax.experimental import pallas as pl
from jax.experimental.pallas import tpu as pltpu
from jax.experimental.pallas import tpu_sc as plsc

G = 8192
D = 256
K = 5
BM = 512
BN = 512
T = G // BN

B = 65536
_NC, _NS = 2, 16
_NW = _NC * _NS
_BPW = B // _NW
_CH = 128
_NCH = _BPW // _CH


def _knn_table_kernel(x_ref, full_ref, out_ref, dist_ref, sqrow_ref,
                      hi_ref, lo_ref):
    pid = pl.program_id(0)

    @pl.when(pid == 0)
    def _():
        ones_row = jnp.ones((1, D), jnp.float32)
        for t in range(T):
            xt = full_ref[pl.ds(t * BN, BN), :]
            sqrow_ref[:, pl.ds(t * BN, BN)] = lax.dot_general(
                ones_row, xt * xt, (((1,), (1,)), ((), ())),
                precision=lax.Precision.HIGHEST,
                preferred_element_type=jnp.float32)
            hi = xt.astype(jnp.bfloat16)
            hi_ref[pl.ds(t * BN, BN), :] = hi
            lo_ref[pl.ds(t * BN, BN), :] = (
                xt - hi.astype(jnp.float32)).astype(jnp.bfloat16)

    x = x_ref[...]
    col_base = lax.broadcasted_iota(jnp.int32, (BM, BN), 1)
    inf = jnp.float32(jnp.inf)
    row_id = pid * BM + lax.broadcasted_iota(jnp.int32, (BM, 1), 0)
    for t in range(T):
        xt = full_ref[pl.ds(t * BN, BN), :]
        dotv = lax.dot_general(
            x, xt, (((1,), (1,)), ((), ())),
            preferred_element_type=jnp.float32)
        sqb = sqrow_ref[:, pl.ds(t * BN, BN)]
        col = col_base + t * BN
        dist_ref[:, pl.ds(t * BN, BN)] = jnp.where(
            col == row_id, inf, sqb - 2.0 * dotv)

    colf_base = col_base.astype(jnp.float32)
    big = jnp.float32(3e8)
    sel = []
    for _ in range(K - 1):
        bestv = jnp.full((BM, 1), inf, jnp.float32)
        besti = jnp.full((BM, 1), big, jnp.float32)
        for t in range(T):
            d = dist_ref[:, pl.ds(t * BN, BN)]
            colf = colf_base + jnp.float32(t * BN)
            for s in sel:
                d = jnp.where(colf == s, inf, d)
            tmin = jnp.min(d, axis=1, keepdims=True)
            targ = jnp.min(jnp.where(d == tmin, colf, big),
                           axis=1, keepdims=True)
            take = tmin < bestv
            bestv = jnp.where(take, tmin, bestv)
            besti = jnp.where(take, targ, besti)
        sel.append(besti)

    acc = jnp.zeros((BM, D), jnp.float32)
    for t in range(T):
        ds = pl.ds(t * BN, BN)
        colf = colf_base + jnp.float32(t * BN)
        oh = ((colf == sel[0]) | (colf == sel[1]) |
              (colf == sel[2]) | (colf == sel[3])).astype(jnp.bfloat16)
        acc = (acc
               + lax.dot_general(oh, hi_ref[ds, :], (((1,), (0,)), ((), ())),
                                 preferred_element_type=jnp.float32)
               + lax.dot_general(oh, lo_ref[ds, :], (((1,), (0,)), ((), ())),
                                 preferred_element_type=jnp.float32))

    s = jnp.sum(acc * acc, axis=1, keepdims=True)
    out_ref[...] = acc / jnp.maximum(jnp.sqrt(s), 1e-12)


def _build_table(x):
    return pl.pallas_call(
        _knn_table_kernel,
        grid=(G // BM,),
        in_specs=[
            pl.BlockSpec((BM, D), lambda i: (i, 0)),
            pl.BlockSpec((G, D), lambda i: (0, 0)),
        ],
        out_specs=pl.BlockSpec((BM, D), lambda i: (i, 0)),
        out_shape=jax.ShapeDtypeStruct((G, D), jnp.float32),
        scratch_shapes=[
            pltpu.VMEM((BM, G), jnp.float32),
            pltpu.VMEM((1, G), jnp.float32),
            pltpu.VMEM((G, D), jnp.bfloat16),
            pltpu.VMEM((G, D), jnp.bfloat16),
        ],
    )(x, x)


def _gather_body(table_hbm, idx_hbm, out_hbm, idx_v, buf0, buf1, sem0, sem1):
    wid = lax.axis_index("s") * _NC + lax.axis_index("c")
    base = wid * _BPW
    pltpu.sync_copy(idx_hbm.at[pl.ds(base, _BPW)], idx_v)
    bufs = (buf0, buf1)
    sems = (sem0, sem1)
    handles = [None, None]
    handles[0] = pltpu.async_copy(
        table_hbm.at[idx_v.at[pl.ds(0, _CH)]], bufs[0], sems[0])
    for c in range(_NCH):
        cur = c % 2
        if c + 1 < _NCH:
            nxt = (c + 1) % 2
            handles[nxt] = pltpu.async_copy(
                table_hbm.at[idx_v.at[pl.ds((c + 1) * _CH, _CH)]],
                bufs[nxt], sems[nxt])
        handles[cur].wait()
        pltpu.sync_copy(bufs[cur], out_hbm.at[pl.ds(base + c * _CH, _CH)])


def _gather_rows(table, idx):
    mesh = plsc.VectorSubcoreMesh(core_axis_name="c", subcore_axis_name="s")
    f = pl.kernel(
        _gather_body,
        out_type=jax.ShapeDtypeStruct((B, D), jnp.float32),
        mesh=mesh,
        scratch_types=[
            pltpu.VMEM((_BPW,), jnp.int32),
            pltpu.VMEM((_CH, D), jnp.float32),
            pltpu.VMEM((_CH, D), jnp.float32),
            pltpu.SemaphoreType.DMA,
            pltpu.SemaphoreType.DMA,
        ],
    )
    return f(table, idx)


def kernel(graph_emb, batch, k):
    table = _build_table(graph_emb)
    return _gather_rows(table, batch.astype(jnp.int32))
